# baseline (speedup 1.0000x reference)
import functools

import jax
import jax.numpy as jnp
from jax.experimental import pallas as pl
from jax.experimental.pallas import tpu as pltpu

TAU = 0.5
K_SEL = 10
B = 128
D = 32768
R = 8
NB = B // R
DC = 256
NCH = D // DC
NEG_INF = float("-inf")


def _sample_body(logits_ref, unif_ref, samples_ref, q_scr):
    l = logits_ref[0]
    e8 = jnp.exp(jnp.minimum(l, 40.0) * 2.0)

    s = jnp.zeros((R, K_SEL, 1), jnp.float32)
    for c in range(NCH):
        w = jnp.log(unif_ref[:, :, pl.ds(c * DC, DC)])
        q = 1.0 / (w * w)
        q_scr[:, :, pl.ds(c * DC, DC)] = q.astype(jnp.bfloat16)
        t = e8[:, None, c * DC:(c + 1) * DC] * q
        s = s + jnp.sum(t, axis=2, keepdims=True)
    rs = 1.0 / s

    for c in range(NCH):
        q = q_scr[:, :, pl.ds(c * DC, DC)].astype(jnp.float32)
        m = jnp.max(q * rs, axis=1)
        samples_ref[0, :, pl.ds(c * DC, DC)] = e8[:, c * DC:(c + 1) * DC] * m


def _topk_body(logits_ref, disc_ref):
    l = logits_ref[0]
    x = l
    remaining = jnp.full((R, 1), K_SEL, jnp.int32)
    thr = jnp.full((R, 1), NEG_INF, jnp.float32)
    for _ in range(K_SEL):
        m = jnp.max(x, axis=1, keepdims=True)
        thr = jnp.where(remaining > 0, m, thr)
        hit = x == m
        c = jnp.sum(jnp.where(hit, 1, 0).astype(jnp.int32), axis=1, keepdims=True)
        remaining = jnp.where(remaining > 0, remaining - c, remaining)
        x = jnp.where(hit, NEG_INF, x)
    disc_ref[0] = (l >= thr).astype(jnp.float32)


@jax.jit
def kernel(logits, uniform):
    logits3 = logits.reshape(NB, R, D)
    samples = pl.pallas_call(
        _sample_body,
        grid=(NB,),
        in_specs=[
            pl.BlockSpec((1, R, D), lambda b: (b, 0, 0)),
            pl.BlockSpec((R, K_SEL, D), lambda b: (b, 0, 0)),
        ],
        out_specs=pl.BlockSpec((1, R, D), lambda b: (b, 0, 0)),
        out_shape=jax.ShapeDtypeStruct((NB, R, D), jnp.float32),
        scratch_shapes=[pltpu.VMEM((R, K_SEL, D), jnp.bfloat16)],
        compiler_params=pltpu.CompilerParams(
            dimension_semantics=("arbitrary",),
        ),
    )(logits3, uniform)
    disc = pl.pallas_call(
        _topk_body,
        grid=(NB,),
        in_specs=[pl.BlockSpec((1, R, D), lambda b: (b, 0, 0))],
        out_specs=pl.BlockSpec((1, R, D), lambda b: (b, 0, 0)),
        out_shape=jax.ShapeDtypeStruct((NB, R, D), jnp.float32),
    )(logits3)
    return samples.reshape(B, D), disc.reshape(B, D)


# speedup vs baseline: 1.0480x; 1.0480x over previous
import functools

import jax
import jax.numpy as jnp
from jax.experimental import pallas as pl
from jax.experimental.pallas import tpu as pltpu

TAU = 0.5
K_SEL = 10
B = 128
D = 32768
R = 8
NB = B // R
DC = 256
NCH = D // DC
NEG_INF = float("-inf")


def _sample_body(logits_ref, unif_ref, samples_ref):
    l = logits_ref[0]
    e8 = jnp.exp(jnp.minimum(l, 40.0) * 2.0)

    s = jnp.zeros((R, K_SEL, 1), jnp.float32)
    for c in range(NCH):
        w = jnp.log(unif_ref[:, :, pl.ds(c * DC, DC)])
        q = 1.0 / (w * w)
        t = e8[:, None, c * DC:(c + 1) * DC] * q
        s = s + jnp.sum(t, axis=2, keepdims=True)
    rs = 1.0 / s

    for c in range(NCH):
        w = jnp.log(unif_ref[:, :, pl.ds(c * DC, DC)])
        q = 1.0 / (w * w)
        m = jnp.max(q * rs, axis=1)
        samples_ref[0, :, pl.ds(c * DC, DC)] = e8[:, c * DC:(c + 1) * DC] * m


def _topk_body(logits_ref, disc_ref):
    l = logits_ref[0]
    x = l
    remaining = jnp.full((R, 1), K_SEL, jnp.int32)
    thr = jnp.full((R, 1), NEG_INF, jnp.float32)
    for _ in range(K_SEL):
        m = jnp.max(x, axis=1, keepdims=True)
        thr = jnp.where(remaining > 0, m, thr)
        hit = x == m
        c = jnp.sum(jnp.where(hit, 1, 0).astype(jnp.int32), axis=1, keepdims=True)
        remaining = jnp.where(remaining > 0, remaining - c, remaining)
        x = jnp.where(hit, NEG_INF, x)
    disc_ref[0] = (l >= thr).astype(jnp.float32)


@jax.jit
def kernel(logits, uniform):
    logits3 = logits.reshape(NB, R, D)
    samples = pl.pallas_call(
        _sample_body,
        grid=(NB,),
        in_specs=[
            pl.BlockSpec((1, R, D), lambda b: (b, 0, 0)),
            pl.BlockSpec((R, K_SEL, D), lambda b: (b, 0, 0)),
        ],
        out_specs=pl.BlockSpec((1, R, D), lambda b: (b, 0, 0)),
        out_shape=jax.ShapeDtypeStruct((NB, R, D), jnp.float32),
        compiler_params=pltpu.CompilerParams(
            dimension_semantics=("arbitrary",),
        ),
    )(logits3, uniform)
    disc = pl.pallas_call(
        _topk_body,
        grid=(NB,),
        in_specs=[pl.BlockSpec((1, R, D), lambda b: (b, 0, 0))],
        out_specs=pl.BlockSpec((1, R, D), lambda b: (b, 0, 0)),
        out_shape=jax.ShapeDtypeStruct((NB, R, D), jnp.float32),
    )(logits3)
    return samples.reshape(B, D), disc.reshape(B, D)
